# baseline (speedup 1.0000x reference)
import functools
import math

import jax
import jax.numpy as jnp
from jax import lax
from jax.experimental import pallas as pl
from jax.experimental.pallas import tpu as pltpu
from jax.experimental.pallas import tpu_sc as plsc

CHUNK = 128
M = 5
LAG = 2


def _scale_block(x_ref, o_ref, *, scale):
    o_ref[...] = x_ref[...] * scale


def kernel(token_ids, token_embed):
    B, L = token_ids.shape
    V, D = token_embed.shape
    scale = math.sqrt(D)
    N = B * L

    TB = 2000
    assert V % TB == 0
    scaled = pl.pallas_call(
        functools.partial(_scale_block, scale=scale),
        grid=(V // TB,),
        in_specs=[pl.BlockSpec((TB, D), lambda i: (i, 0))],
        out_specs=pl.BlockSpec((TB, D), lambda i: (i, 0)),
        out_shape=jax.ShapeDtypeStruct((V, D), jnp.float32),
    )(token_embed)

    info = plsc.get_sparse_core_info()
    NC, NS = info.num_cores, info.num_subcores
    NW = NC * NS
    assert N % (NW * CHUNK) == 0
    per_w = N // NW
    T = per_w // CHUNK
    R = -(-(T + LAG) // M)

    idx = token_ids.reshape(NW, T, CHUNK).astype(jnp.int32)
    mesh = plsc.VectorSubcoreMesh(core_axis_name="c", subcore_axis_name="s")

    @functools.partial(
        pl.kernel,
        mesh=mesh,
        out_type=jax.ShapeDtypeStruct((N, D), jnp.float32),
        scratch_types=[
            pltpu.VMEM((T, CHUNK), jnp.int32),
            pltpu.VMEM((M, CHUNK, D), jnp.float32),
        ] + [pltpu.SemaphoreType.DMA] * (2 * M),
    )
    def sc_gather(idx_hbm, tab_hbm, out_hbm, idx_v, buf, *sems):
        gsem = sems[:M]
        ssem = sems[M:]
        wid = lax.axis_index("s") * NC + lax.axis_index("c")
        row0 = wid * per_w
        pltpu.sync_copy(idx_hbm.at[wid], idx_v)

        def g_start(b, ch):
            pltpu.async_copy(tab_hbm.at[idx_v.at[ch]], buf.at[b], gsem[b])

        def g_wait(b):
            pltpu.make_async_copy(
                tab_hbm.at[pl.ds(0, CHUNK)], buf.at[b], gsem[b]).wait()

        def s_start(b, ch):
            pltpu.async_copy(
                buf.at[b], out_hbm.at[pl.ds(row0 + ch * CHUNK, CHUNK)], ssem[b])

        def s_wait(b):
            pltpu.make_async_copy(
                buf.at[b], out_hbm.at[pl.ds(0, CHUNK)], ssem[b]).wait()

        def round_body(r, carry):
            for b in range(M):
                f = r * M + b

                @pl.when(f < T)
                def _():
                    @pl.when(f >= M)
                    def _():
                        s_wait(b)

                    g_start(b, f)

                ch = f - LAG
                bb = (b - LAG) % M

                @pl.when(jnp.logical_and(ch >= 0, ch < T))
                def _():
                    g_wait(bb)
                    s_start(bb, ch)
            return carry

        lax.fori_loop(0, R, round_body, 0)
        for b in range(M):
            s_wait(b)

    out = sc_gather(idx, scaled)
    return out.reshape(B, L, D)


# speedup vs baseline: 1.0065x; 1.0065x over previous
import functools
import math

import jax
import jax.numpy as jnp
from jax import lax
from jax.experimental import pallas as pl
from jax.experimental.pallas import tpu as pltpu
from jax.experimental.pallas import tpu_sc as plsc

CHUNK = 128
M = 6
LAG = 3


def _scale_block(x_ref, o_ref, *, scale):
    o_ref[...] = x_ref[...] * scale


def kernel(token_ids, token_embed):
    B, L = token_ids.shape
    V, D = token_embed.shape
    scale = math.sqrt(D)
    N = B * L

    TB = 2000
    assert V % TB == 0
    scaled = pl.pallas_call(
        functools.partial(_scale_block, scale=scale),
        grid=(V // TB,),
        in_specs=[pl.BlockSpec((TB, D), lambda i: (i, 0))],
        out_specs=pl.BlockSpec((TB, D), lambda i: (i, 0)),
        out_shape=jax.ShapeDtypeStruct((V, D), jnp.float32),
    )(token_embed)

    info = plsc.get_sparse_core_info()
    NC, NS = info.num_cores, info.num_subcores
    NW = NC * NS
    assert N % (NW * CHUNK) == 0
    per_w = N // NW
    T = per_w // CHUNK
    R = -(-(T + LAG) // M)

    idx = token_ids.reshape(NW, T, CHUNK).astype(jnp.int32)
    mesh = plsc.VectorSubcoreMesh(core_axis_name="c", subcore_axis_name="s")

    @functools.partial(
        pl.kernel,
        mesh=mesh,
        out_type=jax.ShapeDtypeStruct((N, D), jnp.float32),
        scratch_types=[
            pltpu.VMEM((T, CHUNK), jnp.int32),
            pltpu.VMEM((M, CHUNK, D), jnp.float32),
        ] + [pltpu.SemaphoreType.DMA] * (2 * M),
    )
    def sc_gather(idx_hbm, tab_hbm, out_hbm, idx_v, buf, *sems):
        gsem = sems[:M]
        ssem = sems[M:]
        wid = lax.axis_index("s") * NC + lax.axis_index("c")
        row0 = wid * per_w
        pltpu.sync_copy(idx_hbm.at[wid], idx_v)

        def g_start(b, ch):
            pltpu.async_copy(tab_hbm.at[idx_v.at[ch]], buf.at[b], gsem[b])

        def g_wait(b):
            pltpu.make_async_copy(
                tab_hbm.at[pl.ds(0, CHUNK)], buf.at[b], gsem[b]).wait()

        def s_start(b, ch):
            pltpu.async_copy(
                buf.at[b], out_hbm.at[pl.ds(row0 + ch * CHUNK, CHUNK)], ssem[b])

        def s_wait(b):
            pltpu.make_async_copy(
                buf.at[b], out_hbm.at[pl.ds(0, CHUNK)], ssem[b]).wait()

        def round_body(r, carry):
            for b in range(M):
                f = r * M + b

                @pl.when(f < T)
                def _():
                    @pl.when(f >= M)
                    def _():
                        s_wait(b)

                    g_start(b, f)

                ch = f - LAG
                bb = (b - LAG) % M

                @pl.when(jnp.logical_and(ch >= 0, ch < T))
                def _():
                    g_wait(bb)
                    s_start(bb, ch)
            return carry

        lax.fori_loop(0, R, round_body, 0)
        for b in range(M):
            s_wait(b)

    out = sc_gather(idx, scaled)
    return out.reshape(B, L, D)


# speedup vs baseline: 1.1626x; 1.1551x over previous
import functools
import math

import jax
import jax.numpy as jnp
from jax import lax
from jax.experimental import pallas as pl
from jax.experimental.pallas import tpu as pltpu
from jax.experimental.pallas import tpu_sc as plsc

LANES = 16
CHUNK = 128
M = 6
LAG = 3


def kernel(token_ids, token_embed):
    B, L = token_ids.shape
    V, D = token_embed.shape
    scale = math.sqrt(D)
    N = B * L

    info = plsc.get_sparse_core_info()
    NC, NS = info.num_cores, info.num_subcores
    NW = NC * NS
    assert N % (NW * CHUNK) == 0
    per_w = N // NW
    T = per_w // CHUNK
    R = -(-(T + LAG) // M)

    idx = token_ids.reshape(NW, T, CHUNK).astype(jnp.int32)
    mesh = plsc.VectorSubcoreMesh(core_axis_name="c", subcore_axis_name="s")

    @functools.partial(
        pl.kernel,
        mesh=mesh,
        out_type=jax.ShapeDtypeStruct((N, D), jnp.float32),
        scratch_types=[
            pltpu.VMEM((T, CHUNK), jnp.int32),
            pltpu.VMEM((M, CHUNK, D), jnp.float32),
        ] + [pltpu.SemaphoreType.DMA] * (2 * M),
    )
    def sc_gather(idx_hbm, tab_hbm, out_hbm, idx_v, buf, *sems):
        gsem = sems[:M]
        ssem = sems[M:]
        wid = lax.axis_index("s") * NC + lax.axis_index("c")
        row0 = wid * per_w
        pltpu.sync_copy(idx_hbm.at[wid], idx_v)

        def g_start(b, ch):
            pltpu.async_copy(tab_hbm.at[idx_v.at[ch]], buf.at[b], gsem[b])

        def g_wait(b):
            pltpu.make_async_copy(
                tab_hbm.at[pl.ds(0, CHUNK)], buf.at[b], gsem[b]).wait()

        def s_start(b, ch):
            pltpu.async_copy(
                buf.at[b], out_hbm.at[pl.ds(row0 + ch * CHUNK, CHUNK)], ssem[b])

        def s_wait(b):
            pltpu.make_async_copy(
                buf.at[b], out_hbm.at[pl.ds(0, CHUNK)], ssem[b]).wait()

        def round_body(r, carry):
            for b in range(M):
                f = r * M + b

                @pl.when(f < T)
                def _():
                    @pl.when(f >= M)
                    def _():
                        s_wait(b)

                    g_start(b, f)

                ch = f - LAG
                bb = (b - LAG) % M

                @pl.when(jnp.logical_and(ch >= 0, ch < T))
                def _():
                    g_wait(bb)

                    def row_body(rr, c):
                        for g in range(D // LANES):
                            sl = pl.ds(g * LANES, LANES)
                            buf[bb, rr, sl] = buf[bb, rr, sl] * scale
                        return c

                    lax.fori_loop(0, CHUNK, row_body, 0)
                    s_start(bb, ch)
            return carry

        lax.fori_loop(0, R, round_body, 0)
        for b in range(M):
            s_wait(b)

    out = sc_gather(idx, token_embed)
    return out.reshape(B, L, D)
